# baseline (speedup 1.0000x reference)
import jax
import jax.numpy as jnp
from jax import lax
from jax.experimental import pallas as pl
from jax.experimental.pallas import tpu as pltpu

_N = 1024
_DH = 64
_STEPS = 10
_NBLK = 8
_BLK = _N // _NBLK

_T_DIMS = (((0,), (0,)), ((), ()))
_C_DIMS = (((1,), (1,)), ((), ()))


def _leaky(x):
    return jnp.maximum(x, 0.01 * x)


def _body(E_ref, x_ref, W1_ref, b1_ref, ga1_ref, be1_ref,
          W2_ref, b2_ref, ga2_ref, be2_ref,
          Wih_ref, Whh_ref, bih_ref, bhh_ref,
          L1_ref, l1b_ref, L2_ref, l2b_ref, out_ref):
    f32 = jnp.float32
    E = E_ref[...]

    ones = jnp.ones((_N, 1), f32)
    deg = lax.dot_general(E, ones, _T_DIMS, preferred_element_type=f32) + 1.0
    dinv = jnp.where(deg > 0, lax.rsqrt(deg), 0.0)

    def gcn(xin, W_ref, b_ref):
        y = dinv * jnp.dot(xin, W_ref[...], preferred_element_type=f32)
        z = lax.dot_general(E, y, _T_DIMS, preferred_element_type=f32,
                            precision=lax.Precision.HIGHEST) + y
        return dinv * z + b_ref[...].reshape(1, _DH)

    def bn(v, g_ref, b_ref):
        m = jnp.mean(v, axis=0, keepdims=True)
        var = jnp.mean((v - m) ** 2, axis=0, keepdims=True)
        return (g_ref[...].reshape(1, _DH) * (v - m) * lax.rsqrt(var + 1e-5)
                + b_ref[...].reshape(1, _DH))

    h1 = _leaky(bn(gcn(x_ref[...], W1_ref, b1_ref), ga1_ref, be1_ref))
    x2 = _leaky(bn(gcn(h1, W2_ref, b2_ref), ga2_ref, be2_ref) + h1)

    Whx = Wih_ref[:, :_DH] + Whh_ref[...]
    Wir = Wih_ref[:, _DH:]
    M = lax.dot_general(x2, Wir, _C_DIMS,
                        preferred_element_type=f32)
    bgates = (bih_ref[...] + bhh_ref[...]).reshape(1, 4 * _DH)
    h = jnp.zeros((1, _DH), f32)
    c = jnp.zeros((1, _DH), f32)
    ex = jnp.zeros((1, _N), f32)
    s = jnp.ones((1, 1), f32)
    for _ in range(_STEPS):
        gates = (lax.dot_general(h, Whx, _C_DIMS,
                                 preferred_element_type=f32)
                 + jnp.dot(ex, M, preferred_element_type=f32)
                 * lax.reciprocal(s)
                 + bgates)
        i = jax.nn.sigmoid(gates[:, 0:_DH])
        f = jax.nn.sigmoid(gates[:, _DH:2 * _DH])
        g = jnp.tanh(gates[:, 2 * _DH:3 * _DH])
        o = jax.nn.sigmoid(gates[:, 3 * _DH:4 * _DH])
        c = f * c + i * g
        h = o * jnp.tanh(c)
        e = lax.dot_general(h, x2, _C_DIMS,
                            preferred_element_type=f32)
        ex = jnp.exp(e - jnp.max(e, axis=1, keepdims=True))
        s = jnp.sum(ex, axis=1, keepdims=True)

    r = jnp.dot(ex, x2, preferred_element_type=f32) / s
    o1 = _leaky(jnp.dot(h, L1_ref[:_DH], preferred_element_type=f32)
                + jnp.dot(r, L1_ref[_DH:], preferred_element_type=f32)
                + l1b_ref[...].reshape(1, _DH))
    out_ref[...] = (jnp.dot(o1, L2_ref[...], preferred_element_type=f32)
                    + l2b_ref[...].reshape(1, 16))


def _call(*args):
    return pl.pallas_call(
        _body,
        out_shape=jax.ShapeDtypeStruct((1, 16), jnp.float32),
    )(*args)


def kernel(edge_weights, features, W1, b1, gamma1, beta1, W2, b2, gamma2,
           beta2, Wih, Whh, bih, bhh, lin1_W, lin1_b, lin2_W, lin2_b):
    args = (edge_weights, features, W1, b1, gamma1, beta1,
            W2, b2, gamma2, beta2,
            Wih, Whh, bih, bhh,
            lin1_W, lin1_b, lin2_W, lin2_b)
    return _call(*args)


# speedup vs baseline: 1.5073x; 1.5073x over previous
import jax
import jax.numpy as jnp
from jax import lax
from jax.experimental import pallas as pl
from jax.experimental.pallas import tpu as pltpu

_N = 1024
_DH = 64
_STEPS = 10
_NBLK = 8
_BLK = _N // _NBLK

_T_DIMS = (((0,), (0,)), ((), ()))
_C_DIMS = (((1,), (1,)), ((), ()))


def _leaky(x):
    return jnp.maximum(x, 0.01 * x)


def _body(E_ref, x_ref, W1_ref, b1_ref, ga1_ref, be1_ref,
          W2_ref, b2_ref, ga2_ref, be2_ref,
          Wih_ref, Whh_ref, bih_ref, bhh_ref,
          L1_ref, l1b_ref, L2_ref, l2b_ref, out_ref):
    f32 = jnp.float32
    E = E_ref[...]

    ones = jnp.ones((_N, 1), f32)
    deg = lax.dot_general(E, ones, _T_DIMS, preferred_element_type=f32) + 1.0
    dinv = lax.rsqrt(deg)

    def gcn(xin, W_ref, b_ref):
        y = dinv * jnp.dot(xin, W_ref[...], preferred_element_type=f32)
        z = lax.dot_general(E, y, _T_DIMS, preferred_element_type=f32) + y
        return dinv * z + b_ref[...].reshape(1, _DH)

    def bn(v, g_ref, b_ref):
        m = jnp.mean(v, axis=0, keepdims=True)
        var = jnp.mean((v - m) ** 2, axis=0, keepdims=True)
        return (g_ref[...].reshape(1, _DH) * (v - m) * lax.rsqrt(var + 1e-5)
                + b_ref[...].reshape(1, _DH))

    h1 = _leaky(bn(gcn(x_ref[...], W1_ref, b1_ref), ga1_ref, be1_ref))
    x2 = _leaky(bn(gcn(h1, W2_ref, b2_ref), ga2_ref, be2_ref) + h1)

    Whx = Wih_ref[:, :_DH] + Whh_ref[...]
    Wir = Wih_ref[:, _DH:]
    M = lax.dot_general(x2, Wir, _C_DIMS,
                        preferred_element_type=f32)
    bgates = (bih_ref[...] + bhh_ref[...]).reshape(1, 4 * _DH)
    h = jnp.zeros((1, _DH), f32)
    c = jnp.zeros((1, _DH), f32)
    ex = jnp.zeros((1, _N), f32)
    s = jnp.ones((1, 1), f32)
    for _ in range(_STEPS):
        gates = (lax.dot_general(h, Whx, _C_DIMS,
                                 preferred_element_type=f32)
                 + jnp.dot(ex, M, preferred_element_type=f32)
                 * lax.reciprocal(s)
                 + bgates)
        i = jax.nn.sigmoid(gates[:, 0:_DH])
        f = jax.nn.sigmoid(gates[:, _DH:2 * _DH])
        g = jnp.tanh(gates[:, 2 * _DH:3 * _DH])
        o = jax.nn.sigmoid(gates[:, 3 * _DH:4 * _DH])
        c = f * c + i * g
        h = o * jnp.tanh(c)
        e = lax.dot_general(h, x2, _C_DIMS,
                            preferred_element_type=f32)
        ex = jnp.exp(e - jnp.max(e, axis=1, keepdims=True))
        s = jnp.sum(ex, axis=1, keepdims=True)

    r = jnp.dot(ex, x2, preferred_element_type=f32) / s
    o1 = _leaky(jnp.dot(h, L1_ref[:_DH], preferred_element_type=f32)
                + jnp.dot(r, L1_ref[_DH:], preferred_element_type=f32)
                + l1b_ref[...].reshape(1, _DH))
    out_ref[...] = (jnp.dot(o1, L2_ref[...], preferred_element_type=f32)
                    + l2b_ref[...].reshape(1, 16))


def _call(*args):
    return pl.pallas_call(
        _body,
        out_shape=jax.ShapeDtypeStruct((1, 16), jnp.float32),
    )(*args)


def kernel(edge_weights, features, W1, b1, gamma1, beta1, W2, b2, gamma2,
           beta2, Wih, Whh, bih, bhh, lin1_W, lin1_b, lin2_W, lin2_b):
    args = (edge_weights, features, W1, b1, gamma1, beta1,
            W2, b2, gamma2, beta2,
            Wih, Whh, bih, bhh,
            lin1_W, lin1_b, lin2_W, lin2_b)
    return _call(*args)


# speedup vs baseline: 1.5127x; 1.0036x over previous
import jax
import jax.numpy as jnp
from jax import lax
from jax.experimental import pallas as pl

_N = 1024
_DH = 64
_STEPS = 10

_T_DIMS = (((0,), (0,)), ((), ()))
_C_DIMS = (((1,), (1,)), ((), ()))


def _leaky(x):
    return jnp.maximum(x, 0.01 * x)


def _body(E_ref, x_ref, W1_ref, b1_ref, ga1_ref, be1_ref,
          W2_ref, b2_ref, ga2_ref, be2_ref,
          Wih_ref, Whh_ref, bih_ref, bhh_ref,
          L1_ref, l1b_ref, L2_ref, l2b_ref, out_ref):
    f32 = jnp.float32
    E = E_ref[...]

    ones = jnp.ones((_N, 1), f32)
    deg = lax.dot_general(E, ones, _T_DIMS, preferred_element_type=f32) + 1.0
    dinv = lax.rsqrt(deg)

    def gcn(xin, W_ref, b_ref):
        y = dinv * jnp.dot(xin, W_ref[...], preferred_element_type=f32)
        z = lax.dot_general(E, y, _T_DIMS, preferred_element_type=f32) + y
        return dinv * z + b_ref[...].reshape(1, _DH)

    def bn(v, g_ref, b_ref):
        m = jnp.mean(v, axis=0, keepdims=True)
        var = jnp.mean((v - m) ** 2, axis=0, keepdims=True)
        return (g_ref[...].reshape(1, _DH) * (v - m) * lax.rsqrt(var + 1e-5)
                + b_ref[...].reshape(1, _DH))

    h1 = _leaky(bn(gcn(x_ref[...], W1_ref, b1_ref), ga1_ref, be1_ref))
    x2 = _leaky(bn(gcn(h1, W2_ref, b2_ref), ga2_ref, be2_ref) + h1)

    Whx = Wih_ref[:, :_DH] + Whh_ref[...]
    Wir = Wih_ref[:, _DH:]
    M = lax.dot_general(x2, Wir, _C_DIMS,
                        preferred_element_type=f32)
    bgates = (bih_ref[...] + bhh_ref[...]).reshape(1, 4 * _DH)
    h = jnp.zeros((1, _DH), f32)
    c = jnp.zeros((1, _DH), f32)
    ex = jnp.zeros((1, _N), f32)
    s = jnp.ones((1, 1), f32)
    for _ in range(_STEPS):
        gates = (lax.dot_general(h, Whx, _C_DIMS,
                                 preferred_element_type=f32)
                 + jnp.dot(ex, M, preferred_element_type=f32)
                 * lax.reciprocal(s)
                 + bgates)
        i = jax.nn.sigmoid(gates[:, 0:_DH])
        f = jax.nn.sigmoid(gates[:, _DH:2 * _DH])
        g = jnp.tanh(gates[:, 2 * _DH:3 * _DH])
        o = jax.nn.sigmoid(gates[:, 3 * _DH:4 * _DH])
        c = f * c + i * g
        h = o * jnp.tanh(c)
        e = lax.dot_general(h, x2, _C_DIMS,
                            preferred_element_type=f32)
        ex = jnp.exp(e - jnp.max(e, axis=1, keepdims=True))
        s = jnp.sum(ex, axis=1, keepdims=True)

    r = jnp.dot(ex, x2, preferred_element_type=f32) / s
    o1 = _leaky(jnp.dot(h, L1_ref[:_DH], preferred_element_type=f32)
                + jnp.dot(r, L1_ref[_DH:], preferred_element_type=f32)
                + l1b_ref[...].reshape(1, _DH))
    out_ref[...] = (jnp.dot(o1, L2_ref[...], preferred_element_type=f32)
                    + l2b_ref[...].reshape(1, 16))


def _call(*args):
    return pl.pallas_call(
        _body,
        out_shape=jax.ShapeDtypeStruct((1, 16), jnp.float32),
    )(*args)


def kernel(edge_weights, features, W1, b1, gamma1, beta1, W2, b2, gamma2,
           beta2, Wih, Whh, bih, bhh, lin1_W, lin1_b, lin2_W, lin2_b):
    args = (edge_weights, features, W1, b1, gamma1, beta1,
            W2, b2, gamma2, beta2,
            Wih, Whh, bih, bhh,
            lin1_W, lin1_b, lin2_W, lin2_b)
    return _call(*args)
